# Initial kernel scaffold; baseline (speedup 1.0000x reference)
import jax
import jax.numpy as jnp
from jax.experimental import pallas as pl


def kernel(tokens, pad_mask, t, x0):
    raise NotImplementedError("write your pallas kernel here")


# speedup vs baseline: 31.4625x; 31.4625x over previous
import jax
import jax.numpy as jnp
from jax.experimental import pallas as pl

_NODE_ROWS = 16
_EDGE_ROWS = 2040


def _onehot_kernel(x_ref, node_ref, edge_ref):
    xall = x_ref[0].astype(jnp.float32)
    xn16 = xall[:_NODE_ROWS, :]
    xe = xall[_NODE_ROWS:, :]

    i_p = jax.lax.broadcasted_iota(jnp.int32, (256, 16), 0)
    i_a = jax.lax.broadcasted_iota(jnp.int32, (256, 16), 1)
    P = (i_a == i_p // 16).astype(jnp.float32)
    t1 = jax.lax.dot(P, xn16, preferred_element_type=jnp.float32)
    mask = (i_a == i_p % 16).astype(jnp.float32)
    ones = jnp.ones((16, 128), dtype=jnp.float32)
    m = jax.lax.dot(t1 * mask, ones, preferred_element_type=jnp.float32)
    lane = jax.lax.broadcasted_iota(jnp.int32, (1, 128), 1).astype(jnp.float32)
    node_ref[0] = jnp.where(m == lane, 100.0, -100.0)

    s_a = jax.lax.broadcasted_iota(jnp.int32, (16, 128), 0)
    s_l = jax.lax.broadcasted_iota(jnp.int32, (16, 128), 1)
    S = (s_l // 8 == s_a).astype(jnp.float32)
    rep = jax.lax.dot(xe, S, preferred_element_type=jnp.float32)
    vpat = (jax.lax.broadcasted_iota(jnp.int32, (1, 128), 1) % 8).astype(
        jnp.float32
    )
    edge_ref[0] = jnp.where(rep == vpat, 100.0, -100.0)


def kernel(tokens, pad_mask, t, x0):
    B = x0.shape[0]
    xr = x0.reshape(B, _NODE_ROWS + _EDGE_ROWS, 16)
    node, edge_flat = pl.pallas_call(
        _onehot_kernel,
        grid=(B,),
        in_specs=[
            pl.BlockSpec((1, _NODE_ROWS + _EDGE_ROWS, 16), lambda i: (i, 0, 0))
        ],
        out_specs=[
            pl.BlockSpec((1, 256, 128), lambda i: (i, 0, 0)),
            pl.BlockSpec((1, _EDGE_ROWS, 128), lambda i: (i, 0, 0)),
        ],
        out_shape=[
            jax.ShapeDtypeStruct((B, 256, 128), jnp.float32),
            jax.ShapeDtypeStruct((B, _EDGE_ROWS, 128), jnp.float32),
        ],
    )(xr)
    return node, edge_flat.reshape(B, 32640, 8)


# speedup vs baseline: 336.5015x; 10.6953x over previous
import jax
import jax.numpy as jnp
from jax.experimental import pallas as pl

_SEQ = 32896
_N_EDGE = 32640
_GB = 8


def _onehot_kernel(x_ref, node_ref, edge_ref):
    xf = x_ref[0].astype(jnp.float32)

    j_row = jax.lax.broadcasted_iota(jnp.int32, (_GB * 8, _GB), 0)
    j_col = jax.lax.broadcasted_iota(jnp.int32, (_GB * 8, _GB), 1)
    rep = (j_col == j_row // 8).astype(jnp.float32)
    r = jax.lax.dot(rep, xf, preferred_element_type=jnp.float32)
    vrow = (
        jax.lax.broadcasted_iota(jnp.int32, (_GB * 8, 1), 0) % 8
    ).astype(jnp.float32)
    edge = jnp.where(r[:, 256:] == vrow, 100.0, -100.0)
    edge_ref[...] = edge.reshape(_GB, 8, _N_EDGE)

    lane = jax.lax.broadcasted_iota(jnp.int32, (1, 128), 1).astype(jnp.float32)
    ones = jnp.ones((1, 128), dtype=jnp.float32)
    for k in range(_GB):
        xn = xf[k : k + 1, :256]
        col = jax.lax.dot_general(
            xn,
            ones,
            dimension_numbers=(((0,), (0,)), ((), ())),
            preferred_element_type=jnp.float32,
        )
        node_ref[k] = jnp.where(col == lane, 100.0, -100.0)


def kernel(tokens, pad_mask, t, x0):
    B = x0.shape[0]
    xr = x0.reshape(B // _GB, _GB, _SEQ)
    node, edge_vm = pl.pallas_call(
        _onehot_kernel,
        grid=(B // _GB,),
        in_specs=[pl.BlockSpec((1, _GB, _SEQ), lambda i: (i, 0, 0))],
        out_specs=[
            pl.BlockSpec((_GB, 256, 128), lambda i: (i, 0, 0)),
            pl.BlockSpec((_GB, 8, _N_EDGE), lambda i: (i, 0, 0)),
        ],
        out_shape=[
            jax.ShapeDtypeStruct((B, 256, 128), jnp.float32),
            jax.ShapeDtypeStruct((B, 8, _N_EDGE), jnp.float32),
        ],
    )(xr)
    return node, edge_vm.transpose(0, 2, 1)
